# baseline (speedup 1.0000x reference)
import functools

import jax
import jax.numpy as jnp
from jax import lax
from jax.experimental import pallas as pl
from jax.experimental.pallas import tpu as pltpu
from jax.experimental.pallas import tpu_sc as plsc

_N_NODES = 10000
_HIDDEN = 128
_NUM_REL = 18
_NUM_PROTO = 18
_N_TAILS = 1024
_PPAD = 32
_STRIDE = 1040
_A_ROWS = _NUM_REL * _STRIDE
_NC, _NS = 2, 16
_NW = _NC * _NS
_N_EDGES = 320000
_E_PER_W = _N_EDGES // _NW
_ZROWS = _A_ROWS // _NS
_NODE_BLK = 2504
_WP_ROWS = 10016
_SLOT_PAD = 10016
_CAP = 10176


def _node_stage_kernel(x_ref, proto_ref, wp_ref):
    x = x_ref[...]
    pr = proto_ref[...]
    pn = pr / jnp.maximum(
        jnp.sqrt(jnp.sum(pr * pr, axis=1, keepdims=True)), 1e-12)
    xn = x / jnp.maximum(
        jnp.sqrt(jnp.sum(x * x, axis=1, keepdims=True)), 1e-12)
    sims = lax.dot_general(
        xn, pn, (((1,), (1,)), ((), ())),
        precision=lax.Precision.HIGHEST, preferred_element_type=jnp.float32)
    lane = lax.broadcasted_iota(jnp.int32, sims.shape, 1)
    sims = jnp.where(lane < _NUM_PROTO, sims, -1e30)
    m = jnp.max(sims, axis=1, keepdims=True)
    e = jnp.exp(sims - m)
    z = jnp.sum(e, axis=1, keepdims=True)
    p = e / z
    negent = (jnp.sum(p * sims, axis=1, keepdims=True) - m - jnp.log(z))
    row = (pl.program_id(0) * _NODE_BLK
           + lax.broadcasted_iota(jnp.int32, (_NODE_BLK, 1), 0))
    wp_ref[...] = jnp.where(row < _N_NODES, jnp.exp(negent) * p, 0.0)


def _edge_stage_kernel(src_hbm, dst_hbm, rel_hbm, slot_hbm, wp_hbm, zero_hbm,
                       out_hbm, slot_v, src_v, dst_v, rel_v, cj_v, cs_v,
                       cj_row, cs_row, rows_g, acc_sh, sem_i, sem_g, sem_s):
    cid = lax.axis_index("c")
    sid = lax.axis_index("s")
    wid = sid * _NC + cid
    e0 = wid * _E_PER_W
    d_src = pltpu.async_copy(src_hbm.at[pl.ds(e0, _E_PER_W)], src_v, sem_i)
    d_dst = pltpu.async_copy(dst_hbm.at[pl.ds(e0, _E_PER_W)], dst_v, sem_i)
    d_rel = pltpu.async_copy(rel_hbm.at[pl.ds(e0, _E_PER_W)], rel_v, sem_i)
    pltpu.sync_copy(slot_hbm, slot_v)
    pltpu.sync_copy(zero_hbm.at[sid], acc_sh.at[pl.ds(sid * _ZROWS, _ZROWS)])
    d_src.wait()
    d_dst.wait()
    d_rel.wait()
    plsc.subcore_barrier()

    def grp_body(g, count):
        for i in range(4):
            s = pl.ds(g * 64 + i * 16, 16)
            dst16 = dst_v[s]
            slot16 = plsc.load_gather(slot_v, [dst16])
            j16 = rel_v[s] * _STRIDE + slot16
            mask = slot16 < _N_TAILS
            plsc.store_compressed(cj_v.at[pl.ds(count, 16)], j16, mask=mask)
            plsc.store_compressed(cs_v.at[pl.ds(count, 16)], src_v[s],
                                  mask=mask)
            pc = plsc.all_reduce_population_count(mask)
            count = count + pc[0]
        return count

    count = lax.fori_loop(0, _E_PER_W // 64, grp_body, jnp.int32(0))

    zpad = jnp.full((16,), _N_NODES, jnp.int32)
    jpad = jnp.full((16,), _N_TAILS, jnp.int32)
    for t in range(8):
        cs_v[pl.ds(count + 16 * t, 16)] = zpad
        cj_v[pl.ds(count + 16 * t, 16)] = jpad
    ngroups = (count + 127) // 128

    def g_body(g, carry):
        for i in range(8):
            s = pl.ds(i * 16, 16)
            cj_row[s] = cj_v[pl.ds(g * 128 + i * 16, 16)]
            cs_row[s] = cs_v[pl.ds(g * 128 + i * 16, 16)]
        pltpu.async_copy(wp_hbm.at[cs_row], rows_g, sem_g).wait()
        pltpu.sync_copy(rows_g, acc_sh.at[cj_row], add=True)
        return carry

    lax.fori_loop(0, ngroups, g_body, 0)
    plsc.subcore_barrier()
    pltpu.sync_copy(acc_sh.at[pl.ds(sid * _ZROWS, _ZROWS)],
                    out_hbm.at[cid, pl.ds(sid * _ZROWS, _ZROWS)])


_edge_stage = functools.partial(
    pl.kernel,
    out_type=jax.ShapeDtypeStruct((_NC, _A_ROWS, _PPAD), jnp.float32),
    mesh=plsc.VectorSubcoreMesh(core_axis_name="c", subcore_axis_name="s"),
    scratch_types=[
        pltpu.VMEM((_SLOT_PAD,), jnp.int32),
        pltpu.VMEM((_E_PER_W,), jnp.int32),
        pltpu.VMEM((_E_PER_W,), jnp.int32),
        pltpu.VMEM((_E_PER_W,), jnp.int32),
        pltpu.VMEM((_CAP,), jnp.int32),
        pltpu.VMEM((_CAP,), jnp.int32),
        pltpu.VMEM((128,), jnp.int32),
        pltpu.VMEM((128,), jnp.int32),
        pltpu.VMEM((128, _PPAD), jnp.float32),
        pltpu.VMEM_SHARED((_A_ROWS, _PPAD), jnp.float32),
        pltpu.SemaphoreType.DMA,
        pltpu.SemaphoreType.DMA,
        pltpu.SemaphoreType.DMA,
    ],
    compiler_params=pltpu.CompilerParams(
        needs_layout_passes=False, use_tc_tiling_on_sc=False),
)(_edge_stage_kernel)


def _final_stage_kernel(a_ref, m_ref, proto_ref, slot_ref, out_ref):
    a = a_ref[...]
    acc = jnp.zeros((_STRIDE, _HIDDEN), jnp.float32)
    for r in range(_NUM_REL):
        slab = lax.slice(a, (r * _STRIDE, 0), ((r + 1) * _STRIDE, _PPAD))
        acc = acc + lax.dot_general(
            slab, m_ref[r], (((1,), (0,)), ((), ())),
            precision=lax.Precision.DEFAULT,
            preferred_element_type=jnp.float32)
    cols = lax.broadcasted_iota(jnp.int32, (_N_TAILS, _STRIDE), 1)
    oh = (cols == slot_ref[...]).astype(jnp.float32)
    g = lax.dot_general(
        oh, acc, (((1,), (0,)), ((), ())),
        precision=lax.Precision.DEFAULT, preferred_element_type=jnp.float32)
    gn = g / jnp.maximum(
        jnp.sqrt(jnp.sum(g * g, axis=1, keepdims=True)), 1e-12)
    pr = proto_ref[...]
    pn = pr / jnp.maximum(
        jnp.sqrt(jnp.sum(pr * pr, axis=1, keepdims=True)), 1e-12)
    out_ref[...] = lax.dot_general(
        gn, pn, (((1,), (1,)), ((), ())),
        precision=lax.Precision.HIGHEST, preferred_element_type=jnp.float32)


def kernel(input, multi_embedding, proto_embedding, edge_index, edge_type,
           tail_indices):
    x = input.astype(jnp.float32)
    ei = edge_index.astype(jnp.int32)
    et = edge_type.astype(jnp.int32)
    tails = tail_indices.astype(jnp.int32)

    proto_p = jnp.zeros((_PPAD, _HIDDEN), jnp.float32)
    proto_p = proto_p.at[:_NUM_PROTO].set(proto_embedding.astype(jnp.float32))

    wp = pl.pallas_call(
        _node_stage_kernel,
        out_shape=jax.ShapeDtypeStruct((_WP_ROWS, _PPAD), jnp.float32),
        grid=(_WP_ROWS // _NODE_BLK,),
        in_specs=[
            pl.BlockSpec((_NODE_BLK, _HIDDEN), lambda i: (i, 0)),
            pl.BlockSpec((_PPAD, _HIDDEN), lambda i: (0, 0)),
        ],
        out_specs=pl.BlockSpec((_NODE_BLK, _PPAD), lambda i: (i, 0)),
    )(x, proto_p)

    slot_tab = jnp.full((_SLOT_PAD,), _N_TAILS, jnp.int32)
    slot_tab = slot_tab.at[tails].set(jnp.arange(_N_TAILS, dtype=jnp.int32))
    zeros_hbm = jnp.zeros((_NS, _ZROWS, _PPAD), jnp.float32)

    a_out = _edge_stage(ei[0], ei[1], et, slot_tab, wp, zeros_hbm)
    a_sum = a_out[0] + a_out[1]

    m_pad = jnp.zeros((_NUM_REL, _PPAD, _HIDDEN), jnp.float32)
    m_pad = m_pad.at[:, :_NUM_PROTO, :].set(
        multi_embedding.astype(jnp.float32))
    slot_t = slot_tab[tails].reshape(_N_TAILS, 1)

    out = pl.pallas_call(
        _final_stage_kernel,
        out_shape=jax.ShapeDtypeStruct((_N_TAILS, _PPAD), jnp.float32),
    )(a_sum, m_pad, proto_p, slot_t)
    return out[:, :_NUM_PROTO]


# speedup vs baseline: 1.1783x; 1.1783x over previous
import functools

import jax
import jax.numpy as jnp
from jax import lax
from jax.experimental import pallas as pl
from jax.experimental.pallas import tpu as pltpu
from jax.experimental.pallas import tpu_sc as plsc

_N_NODES = 10000
_HIDDEN = 128
_NUM_REL = 18
_NUM_PROTO = 18
_N_TAILS = 1024
_PPAD = 32
_STRIDE = 1040
_A_ROWS = _NUM_REL * _STRIDE
_NC, _NS = 2, 16
_NW = _NC * _NS
_N_EDGES = 320000
_E_PER_W = _N_EDGES // _NW
_ZROWS = _A_ROWS // _NS
_NODE_BLK = 2504
_WP_ROWS = 10016
_SLOT_PAD = 10016
_CAP = 10176


def _node_stage_kernel(x_ref, proto_ref, wp_ref):
    x = x_ref[...]
    pr = proto_ref[...]
    pn = pr / jnp.maximum(
        jnp.sqrt(jnp.sum(pr * pr, axis=1, keepdims=True)), 1e-12)
    xn = x / jnp.maximum(
        jnp.sqrt(jnp.sum(x * x, axis=1, keepdims=True)), 1e-12)
    sims = lax.dot_general(
        xn, pn, (((1,), (1,)), ((), ())),
        precision=lax.Precision.DEFAULT, preferred_element_type=jnp.float32)
    lane = lax.broadcasted_iota(jnp.int32, sims.shape, 1)
    sims = jnp.where(lane < _NUM_PROTO, sims, -1e30)
    m = jnp.max(sims, axis=1, keepdims=True)
    e = jnp.exp(sims - m)
    z = jnp.sum(e, axis=1, keepdims=True)
    p = e / z
    negent = (jnp.sum(p * sims, axis=1, keepdims=True) - m - jnp.log(z))
    row = (pl.program_id(0) * _NODE_BLK
           + lax.broadcasted_iota(jnp.int32, (_NODE_BLK, 1), 0))
    wp_ref[...] = jnp.where(row < _N_NODES, jnp.exp(negent) * p, 0.0)


def _edge_stage_kernel(ei_hbm, rel_hbm, slot_hbm, wp_hbm, zero_hbm,
                       out_hbm, slot_v, src_v, dst_v, rel_v, cj_v, cs_v,
                       cj_row, cs_row, rows_g, acc_sh, sem_i, sem_g, sem_s):
    cid = lax.axis_index("c")
    sid = lax.axis_index("s")
    wid = sid * _NC + cid
    e0 = wid * _E_PER_W
    d_src = pltpu.async_copy(ei_hbm.at[0, pl.ds(e0, _E_PER_W)], src_v, sem_i)
    d_dst = pltpu.async_copy(ei_hbm.at[1, pl.ds(e0, _E_PER_W)], dst_v, sem_i)
    d_rel = pltpu.async_copy(rel_hbm.at[pl.ds(e0, _E_PER_W)], rel_v, sem_i)
    pltpu.sync_copy(slot_hbm, slot_v)
    pltpu.sync_copy(zero_hbm.at[sid], acc_sh.at[pl.ds(sid * _ZROWS, _ZROWS)])
    d_src.wait()
    d_dst.wait()
    d_rel.wait()
    plsc.subcore_barrier()

    def grp_body(g, count):
        for i in range(4):
            s = pl.ds(g * 64 + i * 16, 16)
            dst16 = dst_v[s]
            slot16 = plsc.load_gather(slot_v, [dst16])
            j16 = rel_v[s] * _STRIDE + slot16
            mask = slot16 < _N_TAILS
            plsc.store_compressed(cj_v.at[pl.ds(count, 16)], j16, mask=mask)
            plsc.store_compressed(cs_v.at[pl.ds(count, 16)], src_v[s],
                                  mask=mask)
            pc = plsc.all_reduce_population_count(mask)
            count = count + pc[0]
        return count

    count = lax.fori_loop(0, _E_PER_W // 64, grp_body, jnp.int32(0))

    zpad = jnp.full((16,), _N_NODES, jnp.int32)
    jpad = jnp.full((16,), _N_TAILS, jnp.int32)
    for t in range(8):
        cs_v[pl.ds(count + 16 * t, 16)] = zpad
        cj_v[pl.ds(count + 16 * t, 16)] = jpad
    ngroups = (count + 127) // 128

    def g_body(g, carry):
        for i in range(8):
            s = pl.ds(i * 16, 16)
            cj_row[s] = cj_v[pl.ds(g * 128 + i * 16, 16)]
            cs_row[s] = cs_v[pl.ds(g * 128 + i * 16, 16)]
        pltpu.async_copy(wp_hbm.at[cs_row], rows_g, sem_g).wait()
        pltpu.sync_copy(rows_g, acc_sh.at[cj_row], add=True)
        return carry

    lax.fori_loop(0, ngroups, g_body, 0)
    plsc.subcore_barrier()
    pltpu.sync_copy(acc_sh.at[pl.ds(sid * _ZROWS, _ZROWS)],
                    out_hbm.at[cid, pl.ds(sid * _ZROWS, _ZROWS)])


_edge_stage = functools.partial(
    pl.kernel,
    out_type=jax.ShapeDtypeStruct((_NC, _A_ROWS, _PPAD), jnp.float32),
    mesh=plsc.VectorSubcoreMesh(core_axis_name="c", subcore_axis_name="s"),
    scratch_types=[
        pltpu.VMEM((_SLOT_PAD,), jnp.int32),
        pltpu.VMEM((_E_PER_W,), jnp.int32),
        pltpu.VMEM((_E_PER_W,), jnp.int32),
        pltpu.VMEM((_E_PER_W,), jnp.int32),
        pltpu.VMEM((_CAP,), jnp.int32),
        pltpu.VMEM((_CAP,), jnp.int32),
        pltpu.VMEM((128,), jnp.int32),
        pltpu.VMEM((128,), jnp.int32),
        pltpu.VMEM((128, _PPAD), jnp.float32),
        pltpu.VMEM_SHARED((_A_ROWS, _PPAD), jnp.float32),
        pltpu.SemaphoreType.DMA,
        pltpu.SemaphoreType.DMA,
        pltpu.SemaphoreType.DMA,
    ],
    compiler_params=pltpu.CompilerParams(
        needs_layout_passes=False, use_tc_tiling_on_sc=False),
)(_edge_stage_kernel)


def _final_stage_kernel(a_hbm, m_ref, proto_ref, slot_ref, out_ref, a_vmem,
                        sem_a):
    pltpu.async_copy(a_hbm, a_vmem, sem_a).wait()
    a = a_vmem[0] + a_vmem[1]
    acc = jnp.zeros((_STRIDE, _HIDDEN), jnp.float32)
    for r in range(_NUM_REL):
        slab = lax.slice(a, (r * _STRIDE, 0), ((r + 1) * _STRIDE, _PPAD))
        acc = acc + lax.dot_general(
            slab, m_ref[r], (((1,), (0,)), ((), ())),
            precision=lax.Precision.DEFAULT,
            preferred_element_type=jnp.float32)
    cols = lax.broadcasted_iota(jnp.int32, (_N_TAILS, _STRIDE), 1)
    oh = (cols == slot_ref[...]).astype(jnp.float32)
    g = lax.dot_general(
        oh, acc, (((1,), (0,)), ((), ())),
        precision=lax.Precision.DEFAULT, preferred_element_type=jnp.float32)
    gn = g / jnp.maximum(
        jnp.sqrt(jnp.sum(g * g, axis=1, keepdims=True)), 1e-12)
    pr = proto_ref[...]
    pn = pr / jnp.maximum(
        jnp.sqrt(jnp.sum(pr * pr, axis=1, keepdims=True)), 1e-12)
    out_ref[...] = lax.dot_general(
        gn, pn, (((1,), (1,)), ((), ())),
        precision=lax.Precision.HIGHEST, preferred_element_type=jnp.float32)


def kernel(input, multi_embedding, proto_embedding, edge_index, edge_type,
           tail_indices):
    x = input.astype(jnp.float32)
    ei = edge_index.astype(jnp.int32)
    et = edge_type.astype(jnp.int32)
    tails = tail_indices.astype(jnp.int32)

    proto_p = jnp.zeros((_PPAD, _HIDDEN), jnp.float32)
    proto_p = proto_p.at[:_NUM_PROTO].set(proto_embedding.astype(jnp.float32))

    wp = pl.pallas_call(
        _node_stage_kernel,
        out_shape=jax.ShapeDtypeStruct((_WP_ROWS, _PPAD), jnp.float32),
        grid=(_WP_ROWS // _NODE_BLK,),
        in_specs=[
            pl.BlockSpec((_NODE_BLK, _HIDDEN), lambda i: (i, 0)),
            pl.BlockSpec((_PPAD, _HIDDEN), lambda i: (0, 0)),
        ],
        out_specs=pl.BlockSpec((_NODE_BLK, _PPAD), lambda i: (i, 0)),
    )(x, proto_p)

    slot_tab = jnp.full((_SLOT_PAD,), _N_TAILS, jnp.int32)
    slot_tab = slot_tab.at[tails].set(jnp.arange(_N_TAILS, dtype=jnp.int32))
    zeros_hbm = jnp.zeros((_NS, _ZROWS, _PPAD), jnp.float32)

    a_out = _edge_stage(ei, et, slot_tab, wp, zeros_hbm)

    m_pad = jnp.zeros((_NUM_REL, _PPAD, _HIDDEN), jnp.float32)
    m_pad = m_pad.at[:, :_NUM_PROTO, :].set(
        multi_embedding.astype(jnp.float32))
    slot_t = slot_tab[tails].reshape(_N_TAILS, 1)

    out = pl.pallas_call(
        _final_stage_kernel,
        out_shape=jax.ShapeDtypeStruct((_N_TAILS, _PPAD), jnp.float32),
        in_specs=[
            pl.BlockSpec(memory_space=pl.ANY),
            pl.BlockSpec(memory_space=pltpu.VMEM),
            pl.BlockSpec(memory_space=pltpu.VMEM),
            pl.BlockSpec(memory_space=pltpu.VMEM),
        ],
        scratch_shapes=[
            pltpu.VMEM((_NC, _A_ROWS, _PPAD), jnp.float32),
            pltpu.SemaphoreType.DMA,
        ],
    )(a_out, m_pad, proto_p, slot_t)
    return out[:, :_NUM_PROTO]


# speedup vs baseline: 1.2523x; 1.0628x over previous
import functools

import jax
import jax.numpy as jnp
from jax import lax
from jax.experimental import pallas as pl
from jax.experimental.pallas import tpu as pltpu
from jax.experimental.pallas import tpu_sc as plsc

_N_NODES = 10000
_HIDDEN = 128
_NUM_REL = 18
_NUM_PROTO = 18
_N_TAILS = 1024
_PPAD = 32
_STRIDE = 1040
_A_ROWS = _NUM_REL * _STRIDE
_NC, _NS = 2, 16
_NW = _NC * _NS
_N_EDGES = 320000
_E_PER_W = _N_EDGES // _NW
_ZROWS = _A_ROWS // _NS
_NODE_BLK = 2504
_WP_ROWS = 10016
_SLOT_PAD = 10016
_CAP = 10176


def _node_stage_kernel(x_ref, proto_ref, wp_ref):
    x = x_ref[...]
    pr = proto_ref[...]
    pn = pr / jnp.maximum(
        jnp.sqrt(jnp.sum(pr * pr, axis=1, keepdims=True)), 1e-12)
    xn = x / jnp.maximum(
        jnp.sqrt(jnp.sum(x * x, axis=1, keepdims=True)), 1e-12)
    sims = lax.dot_general(
        xn, pn, (((1,), (1,)), ((), ())),
        precision=lax.Precision.DEFAULT, preferred_element_type=jnp.float32)
    lane = lax.broadcasted_iota(jnp.int32, sims.shape, 1)
    sims = jnp.where(lane < _NUM_PROTO, sims, -1e30)
    m = jnp.max(sims, axis=1, keepdims=True)
    e = jnp.exp(sims - m)
    z = jnp.sum(e, axis=1, keepdims=True)
    p = e / z
    negent = (jnp.sum(p * sims, axis=1, keepdims=True) - m - jnp.log(z))
    row = (pl.program_id(0) * _NODE_BLK
           + lax.broadcasted_iota(jnp.int32, (_NODE_BLK, 1), 0))
    wp_ref[...] = jnp.where(row < _N_NODES, jnp.exp(negent) * p, 0.0)


def _edge_stage_kernel(ei_hbm, rel_hbm, slot_hbm, wp_hbm, zero_hbm,
                       out_hbm, slot_v, src_v, dst_v, rel_v, cp_v,
                       cj_row, cs_row, rows_g, acc_sh, sem_i, sem_g, sem_s):
    cid = lax.axis_index("c")
    sid = lax.axis_index("s")
    wid = sid * _NC + cid
    e0 = wid * _E_PER_W
    d_src = pltpu.async_copy(ei_hbm.at[0, pl.ds(e0, _E_PER_W)], src_v, sem_i)
    d_dst = pltpu.async_copy(ei_hbm.at[1, pl.ds(e0, _E_PER_W)], dst_v, sem_i)
    d_rel = pltpu.async_copy(rel_hbm.at[pl.ds(e0, _E_PER_W)], rel_v, sem_i)
    pltpu.sync_copy(slot_hbm, slot_v)
    pltpu.sync_copy(zero_hbm.at[sid], acc_sh.at[pl.ds(sid * _ZROWS, _ZROWS)])
    d_src.wait()
    d_dst.wait()
    d_rel.wait()
    plsc.subcore_barrier()

    _UNROLL = 25

    def grp_body(g, count):
        vals, masks = [], []
        for i in range(_UNROLL):
            s = pl.ds(g * (16 * _UNROLL) + i * 16, 16)
            slot16 = plsc.load_gather(slot_v, [dst_v[s]])
            j16 = rel_v[s] * _STRIDE + slot16
            vals.append(j16 * 16384 + src_v[s])
            masks.append(slot16 < _N_TAILS)
        for i in range(_UNROLL):
            plsc.store_compressed(cp_v.at[pl.ds(count, 16)], vals[i],
                                  mask=masks[i])
            count = count + plsc.all_reduce_population_count(masks[i])[0]
        return count

    count = lax.fori_loop(0, _E_PER_W // (16 * _UNROLL), grp_body,
                          jnp.int32(0))

    vpad = jnp.full((16,), _N_TAILS * 16384 + _N_NODES, jnp.int32)
    for t in range(8):
        cp_v[pl.ds(count + 16 * t, 16)] = vpad
    ngroups = (count + 127) // 128

    def g_body(g, carry):
        for i in range(8):
            s = pl.ds(i * 16, 16)
            v16 = cp_v[pl.ds(g * 128 + i * 16, 16)]
            cs_row[s] = lax.bitwise_and(v16, 16383)
            cj_row[s] = lax.shift_right_logical(v16, 14)
        pltpu.async_copy(wp_hbm.at[cs_row], rows_g, sem_g).wait()
        pltpu.sync_copy(rows_g, acc_sh.at[cj_row], add=True)
        return carry

    lax.fori_loop(0, ngroups, g_body, 0)
    plsc.subcore_barrier()
    pltpu.sync_copy(acc_sh.at[pl.ds(sid * _ZROWS, _ZROWS)],
                    out_hbm.at[cid, pl.ds(sid * _ZROWS, _ZROWS)])


_edge_stage = functools.partial(
    pl.kernel,
    out_type=jax.ShapeDtypeStruct((_NC, _A_ROWS, _PPAD), jnp.float32),
    mesh=plsc.VectorSubcoreMesh(core_axis_name="c", subcore_axis_name="s"),
    scratch_types=[
        pltpu.VMEM((_SLOT_PAD,), jnp.int32),
        pltpu.VMEM((_E_PER_W,), jnp.int32),
        pltpu.VMEM((_E_PER_W,), jnp.int32),
        pltpu.VMEM((_E_PER_W,), jnp.int32),
        pltpu.VMEM((_CAP,), jnp.int32),
        pltpu.VMEM((128,), jnp.int32),
        pltpu.VMEM((128,), jnp.int32),
        pltpu.VMEM((128, _PPAD), jnp.float32),
        pltpu.VMEM_SHARED((_A_ROWS, _PPAD), jnp.float32),
        pltpu.SemaphoreType.DMA,
        pltpu.SemaphoreType.DMA,
        pltpu.SemaphoreType.DMA,
    ],
    compiler_params=pltpu.CompilerParams(
        needs_layout_passes=False, use_tc_tiling_on_sc=False),
)(_edge_stage_kernel)


def _final_stage_kernel(a_hbm, m_ref, proto_ref, slot_ref, out_ref, a_vmem,
                        sem_a):
    pltpu.async_copy(a_hbm, a_vmem, sem_a).wait()
    a = a_vmem[0] + a_vmem[1]
    acc = jnp.zeros((_STRIDE, _HIDDEN), jnp.float32)
    for r in range(_NUM_REL):
        slab = lax.slice(a, (r * _STRIDE, 0), ((r + 1) * _STRIDE, _PPAD))
        acc = acc + lax.dot_general(
            slab, m_ref[r], (((1,), (0,)), ((), ())),
            precision=lax.Precision.DEFAULT,
            preferred_element_type=jnp.float32)
    cols = lax.broadcasted_iota(jnp.int32, (_N_TAILS, _STRIDE), 1)
    oh = (cols == slot_ref[...]).astype(jnp.float32)
    g = lax.dot_general(
        oh, acc, (((1,), (0,)), ((), ())),
        precision=lax.Precision.DEFAULT, preferred_element_type=jnp.float32)
    gn = g / jnp.maximum(
        jnp.sqrt(jnp.sum(g * g, axis=1, keepdims=True)), 1e-12)
    pr = proto_ref[...]
    pn = pr / jnp.maximum(
        jnp.sqrt(jnp.sum(pr * pr, axis=1, keepdims=True)), 1e-12)
    out_ref[...] = lax.dot_general(
        gn, pn, (((1,), (1,)), ((), ())),
        precision=lax.Precision.HIGHEST, preferred_element_type=jnp.float32)


def kernel(input, multi_embedding, proto_embedding, edge_index, edge_type,
           tail_indices):
    x = input.astype(jnp.float32)
    ei = edge_index.astype(jnp.int32)
    et = edge_type.astype(jnp.int32)
    tails = tail_indices.astype(jnp.int32)

    proto_p = jnp.zeros((_PPAD, _HIDDEN), jnp.float32)
    proto_p = proto_p.at[:_NUM_PROTO].set(proto_embedding.astype(jnp.float32))

    wp = pl.pallas_call(
        _node_stage_kernel,
        out_shape=jax.ShapeDtypeStruct((_WP_ROWS, _PPAD), jnp.float32),
        grid=(_WP_ROWS // _NODE_BLK,),
        in_specs=[
            pl.BlockSpec((_NODE_BLK, _HIDDEN), lambda i: (i, 0)),
            pl.BlockSpec((_PPAD, _HIDDEN), lambda i: (0, 0)),
        ],
        out_specs=pl.BlockSpec((_NODE_BLK, _PPAD), lambda i: (i, 0)),
    )(x, proto_p)

    slot_tab = jnp.full((_SLOT_PAD,), _N_TAILS, jnp.int32)
    slot_tab = slot_tab.at[tails].set(jnp.arange(_N_TAILS, dtype=jnp.int32))
    zeros_hbm = jnp.zeros((_NS, _ZROWS, _PPAD), jnp.float32)

    a_out = _edge_stage(ei, et, slot_tab, wp, zeros_hbm)

    m_pad = jnp.zeros((_NUM_REL, _PPAD, _HIDDEN), jnp.float32)
    m_pad = m_pad.at[:, :_NUM_PROTO, :].set(
        multi_embedding.astype(jnp.float32))
    slot_t = slot_tab[tails].reshape(_N_TAILS, 1)

    out = pl.pallas_call(
        _final_stage_kernel,
        out_shape=jax.ShapeDtypeStruct((_N_TAILS, _PPAD), jnp.float32),
        in_specs=[
            pl.BlockSpec(memory_space=pl.ANY),
            pl.BlockSpec(memory_space=pltpu.VMEM),
            pl.BlockSpec(memory_space=pltpu.VMEM),
            pl.BlockSpec(memory_space=pltpu.VMEM),
        ],
        scratch_shapes=[
            pltpu.VMEM((_NC, _A_ROWS, _PPAD), jnp.float32),
            pltpu.SemaphoreType.DMA,
        ],
    )(a_out, m_pad, proto_p, slot_t)
    return out[:, :_NUM_PROTO]


# speedup vs baseline: 1.4655x; 1.1703x over previous
import functools

import jax
import jax.numpy as jnp
from jax import lax
from jax.experimental import pallas as pl
from jax.experimental.pallas import tpu as pltpu
from jax.experimental.pallas import tpu_sc as plsc

_N_NODES = 10000
_HIDDEN = 128
_NUM_REL = 18
_NUM_PROTO = 18
_N_TAILS = 1024
_PPAD = 32
_STRIDE = 1040
_A_ROWS = _NUM_REL * _STRIDE
_NC, _NS = 2, 16
_NW = _NC * _NS
_N_EDGES = 320000
_E_PER_W = _N_EDGES // _NW
_ZROWS = _A_ROWS // _NS
_NODE_BLK = 2504
_WP_ROWS = 10016
_SLOT_PAD = 10016
_CAP = 10176


def _node_stage_kernel(x_ref, proto_ref, wp_ref):
    x = x_ref[...]
    pr = proto_ref[...]
    pn = pr / jnp.maximum(
        jnp.sqrt(jnp.sum(pr * pr, axis=1, keepdims=True)), 1e-12)
    xn = x / jnp.maximum(
        jnp.sqrt(jnp.sum(x * x, axis=1, keepdims=True)), 1e-12)
    sims = lax.dot_general(
        xn, pn, (((1,), (1,)), ((), ())),
        precision=lax.Precision.DEFAULT, preferred_element_type=jnp.float32)
    lane = lax.broadcasted_iota(jnp.int32, sims.shape, 1)
    sims = jnp.where(lane < _NUM_PROTO, sims, -1e30)
    m = jnp.max(sims, axis=1, keepdims=True)
    e = jnp.exp(sims - m)
    z = jnp.sum(e, axis=1, keepdims=True)
    p = e / z
    negent = (jnp.sum(p * sims, axis=1, keepdims=True) - m - jnp.log(z))
    row = (pl.program_id(0) * _NODE_BLK
           + lax.broadcasted_iota(jnp.int32, (_NODE_BLK, 1), 0))
    wp_ref[...] = jnp.where(row < _N_NODES, jnp.exp(negent) * p, 0.0)


def _edge_stage_kernel(ei_hbm, rel_hbm, slot_hbm, wp_hbm, zero_hbm,
                       out_hbm, slot_v, src_v, dst_v, rel_v, cp_v,
                       cj_row, cs_row, rows_g, acc_sh, sem_i, sem_g, sem_s):
    cid = lax.axis_index("c")
    sid = lax.axis_index("s")
    wid = sid * _NC + cid
    e0 = wid * _E_PER_W
    d_src = pltpu.async_copy(ei_hbm.at[0, pl.ds(e0, _E_PER_W)], src_v, sem_i)
    d_dst = pltpu.async_copy(ei_hbm.at[1, pl.ds(e0, _E_PER_W)], dst_v, sem_i)
    d_rel = pltpu.async_copy(rel_hbm.at[pl.ds(e0, _E_PER_W)], rel_v, sem_i)
    pltpu.sync_copy(slot_hbm, slot_v)
    pltpu.sync_copy(zero_hbm.at[sid], acc_sh.at[pl.ds(sid * _ZROWS, _ZROWS)])
    d_src.wait()
    d_dst.wait()
    d_rel.wait()
    plsc.subcore_barrier()

    _UNROLL = 25

    def grp_body(g, count):
        vals, masks = [], []
        for i in range(_UNROLL):
            s = pl.ds(g * (16 * _UNROLL) + i * 16, 16)
            slot16 = plsc.load_gather(slot_v, [dst_v[s]])
            j16 = rel_v[s] * _STRIDE + slot16
            vals.append(j16 * 16384 + src_v[s])
            masks.append(slot16 < _N_TAILS)
        for i in range(_UNROLL):
            plsc.store_compressed(cp_v.at[pl.ds(count, 16)], vals[i],
                                  mask=masks[i])
            count = count + plsc.all_reduce_population_count(masks[i])[0]
        return count

    count = lax.fori_loop(0, _E_PER_W // (16 * _UNROLL), grp_body,
                          jnp.int32(0))

    vpad = jnp.full((16,), _N_TAILS * 16384 + _N_NODES, jnp.int32)
    for t in range(8):
        cp_v[pl.ds(count + 16 * t, 16)] = vpad
    ngroups = (count + 127) // 128

    def g_body(g, carry):
        for i in range(8):
            s = pl.ds(i * 16, 16)
            v16 = cp_v[pl.ds(g * 128 + i * 16, 16)]
            cs_row[s] = lax.bitwise_and(v16, 16383)
            cj_row[s] = lax.shift_right_logical(v16, 14)
        pltpu.async_copy(wp_hbm.at[cs_row], rows_g, sem_g).wait()
        pltpu.sync_copy(rows_g, acc_sh.at[cj_row], add=True)
        return carry

    lax.fori_loop(0, ngroups, g_body, 0)
    plsc.subcore_barrier()
    pltpu.sync_copy(acc_sh.at[pl.ds(sid * _ZROWS, _ZROWS)],
                    out_hbm.at[cid, pl.ds(sid * _ZROWS, _ZROWS)])


_edge_stage = functools.partial(
    pl.kernel,
    out_type=jax.ShapeDtypeStruct((_NC, _A_ROWS, _PPAD), jnp.float32),
    mesh=plsc.VectorSubcoreMesh(core_axis_name="c", subcore_axis_name="s"),
    scratch_types=[
        pltpu.VMEM((_SLOT_PAD,), jnp.int32),
        pltpu.VMEM((_E_PER_W,), jnp.int32),
        pltpu.VMEM((_E_PER_W,), jnp.int32),
        pltpu.VMEM((_E_PER_W,), jnp.int32),
        pltpu.VMEM((_CAP,), jnp.int32),
        pltpu.VMEM((128,), jnp.int32),
        pltpu.VMEM((128,), jnp.int32),
        pltpu.VMEM((128, _PPAD), jnp.float32),
        pltpu.VMEM_SHARED((_A_ROWS, _PPAD), jnp.float32),
        pltpu.SemaphoreType.DMA,
        pltpu.SemaphoreType.DMA,
        pltpu.SemaphoreType.DMA,
    ],
    compiler_params=pltpu.CompilerParams(
        needs_layout_passes=False, use_tc_tiling_on_sc=False),
)(_edge_stage_kernel)


def _final_stage_kernel(a_hbm, m_ref, proto_ref, slot_ref, out_ref, a_vmem,
                        sem_a):
    pltpu.async_copy(a_hbm, a_vmem, sem_a).wait()
    a = a_vmem[0] + a_vmem[1]
    qrows = _STRIDE // 4
    slot = slot_ref[...]
    cols = lax.broadcasted_iota(jnp.int32, (_N_TAILS, qrows), 1)
    q = lax.shift_right_logical(slot, 1 + 1)
    u_b = lax.bitwise_and(slot, 3)
    g = jnp.zeros((_N_TAILS, _HIDDEN), jnp.float32)
    for u in range(4):
        acc = jnp.zeros((qrows, _HIDDEN), jnp.float32)
        for r in range(_NUM_REL):
            slab = lax.slice(a, (r * qrows, u * _PPAD),
                             ((r + 1) * qrows, (u + 1) * _PPAD))
            acc = acc + lax.dot_general(
                slab, m_ref[r], (((1,), (0,)), ((), ())),
                precision=lax.Precision.DEFAULT,
                preferred_element_type=jnp.float32)
        oh = ((cols == q) & (u_b == u)).astype(jnp.float32)
        g = g + lax.dot_general(
            oh, acc, (((1,), (0,)), ((), ())),
            precision=lax.Precision.DEFAULT,
            preferred_element_type=jnp.float32)
    gn = g / jnp.maximum(
        jnp.sqrt(jnp.sum(g * g, axis=1, keepdims=True)), 1e-12)
    pr = proto_ref[...]
    pn = pr / jnp.maximum(
        jnp.sqrt(jnp.sum(pr * pr, axis=1, keepdims=True)), 1e-12)
    out_ref[...] = lax.dot_general(
        gn, pn, (((1,), (1,)), ((), ())),
        precision=lax.Precision.HIGHEST, preferred_element_type=jnp.float32)


def kernel(input, multi_embedding, proto_embedding, edge_index, edge_type,
           tail_indices):
    x = input.astype(jnp.float32)
    ei = edge_index.astype(jnp.int32)
    et = edge_type.astype(jnp.int32)
    tails = tail_indices.astype(jnp.int32)

    proto_p = jnp.zeros((_PPAD, _HIDDEN), jnp.float32)
    proto_p = proto_p.at[:_NUM_PROTO].set(proto_embedding.astype(jnp.float32))

    wp = pl.pallas_call(
        _node_stage_kernel,
        out_shape=jax.ShapeDtypeStruct((_WP_ROWS, _PPAD), jnp.float32),
        grid=(_WP_ROWS // _NODE_BLK,),
        in_specs=[
            pl.BlockSpec((_NODE_BLK, _HIDDEN), lambda i: (i, 0)),
            pl.BlockSpec((_PPAD, _HIDDEN), lambda i: (0, 0)),
        ],
        out_specs=pl.BlockSpec((_NODE_BLK, _PPAD), lambda i: (i, 0)),
    )(x, proto_p)

    slot_tab = jnp.full((_SLOT_PAD,), _N_TAILS, jnp.int32)
    slot_tab = slot_tab.at[tails].set(jnp.arange(_N_TAILS, dtype=jnp.int32))
    zeros_hbm = jnp.zeros((_NS, _ZROWS, _PPAD), jnp.float32)

    a_out = _edge_stage(ei, et, slot_tab, wp, zeros_hbm)

    m_pad = jnp.zeros((_NUM_REL, _PPAD, _HIDDEN), jnp.float32)
    m_pad = m_pad.at[:, :_NUM_PROTO, :].set(
        multi_embedding.astype(jnp.float32))
    slot_t = slot_tab[tails].reshape(_N_TAILS, 1)

    out = pl.pallas_call(
        _final_stage_kernel,
        out_shape=jax.ShapeDtypeStruct((_N_TAILS, _PPAD), jnp.float32),
        in_specs=[
            pl.BlockSpec(memory_space=pl.ANY),
            pl.BlockSpec(memory_space=pltpu.VMEM),
            pl.BlockSpec(memory_space=pltpu.VMEM),
            pl.BlockSpec(memory_space=pltpu.VMEM),
        ],
        scratch_shapes=[
            pltpu.VMEM((_NC, _A_ROWS // 4, 128), jnp.float32),
            pltpu.SemaphoreType.DMA,
        ],
    )(a_out.reshape(_NC, _A_ROWS // 4, 128), m_pad, proto_p, slot_t)
    return out[:, :_NUM_PROTO]


# speedup vs baseline: 1.4675x; 1.0013x over previous
import functools

import jax
import jax.numpy as jnp
from jax import lax
from jax.experimental import pallas as pl
from jax.experimental.pallas import tpu as pltpu
from jax.experimental.pallas import tpu_sc as plsc

_N_NODES = 10000
_HIDDEN = 128
_NUM_REL = 18
_NUM_PROTO = 18
_N_TAILS = 1024
_PPAD = 32
_STRIDE = 1040
_A_ROWS = _NUM_REL * _STRIDE
_NC, _NS = 2, 16
_NW = _NC * _NS
_N_EDGES = 320000
_E_PER_W = _N_EDGES // _NW
_ZROWS = _A_ROWS // _NS
_NODE_BLK = 2504
_WP_ROWS = 10016
_SLOT_PAD = 10016
_CAP = 10176


def _node_stage_kernel(x_ref, proto_ref, wp_ref):
    x = x_ref[...]
    pr = proto_ref[...]
    pn = pr / jnp.maximum(
        jnp.sqrt(jnp.sum(pr * pr, axis=1, keepdims=True)), 1e-12)
    xn = x / jnp.maximum(
        jnp.sqrt(jnp.sum(x * x, axis=1, keepdims=True)), 1e-12)
    sims = lax.dot_general(
        xn, pn, (((1,), (1,)), ((), ())),
        precision=lax.Precision.DEFAULT, preferred_element_type=jnp.float32)
    lane = lax.broadcasted_iota(jnp.int32, sims.shape, 1)
    sims = jnp.where(lane < _NUM_PROTO, sims, -1e30)
    m = jnp.max(sims, axis=1, keepdims=True)
    e = jnp.exp(sims - m)
    z = jnp.sum(e, axis=1, keepdims=True)
    p = e / z
    negent = (jnp.sum(p * sims, axis=1, keepdims=True) - m - jnp.log(z))
    row = (pl.program_id(0) * _NODE_BLK
           + lax.broadcasted_iota(jnp.int32, (_NODE_BLK, 1), 0))
    wp_ref[...] = jnp.where(row < _N_NODES, jnp.exp(negent) * p, 0.0)


def _edge_stage_kernel(ei_hbm, rel_hbm, slot_hbm, wp_hbm, zero_hbm,
                       out_hbm, slot_v, src_v, dst_v, rel_v, cp_v,
                       cj_row, cs_row, rows_g, acc_sh, sem_i, sem_g, sem_s):
    cid = lax.axis_index("c")
    sid = lax.axis_index("s")
    wid = sid * _NC + cid
    e0 = wid * _E_PER_W
    d_src = pltpu.async_copy(ei_hbm.at[0, pl.ds(e0, _E_PER_W)], src_v, sem_i)
    d_dst = pltpu.async_copy(ei_hbm.at[1, pl.ds(e0, _E_PER_W)], dst_v, sem_i)
    d_rel = pltpu.async_copy(rel_hbm.at[pl.ds(e0, _E_PER_W)], rel_v, sem_i)
    pltpu.sync_copy(slot_hbm, slot_v)
    pltpu.sync_copy(zero_hbm.at[sid], acc_sh.at[pl.ds(sid * _ZROWS, _ZROWS)])
    d_src.wait()
    d_dst.wait()
    d_rel.wait()
    plsc.subcore_barrier()

    _UNROLL = 25

    def grp_body(g, count):
        vals, masks = [], []
        for i in range(_UNROLL):
            s = pl.ds(g * (16 * _UNROLL) + i * 16, 16)
            slot16 = plsc.load_gather(slot_v, [dst_v[s]])
            j16 = rel_v[s] * _STRIDE + slot16
            vals.append(j16 * 16384 + src_v[s])
            masks.append(slot16 < _N_TAILS)
        for i in range(_UNROLL):
            plsc.store_compressed(cp_v.at[pl.ds(count, 16)], vals[i],
                                  mask=masks[i])
            count = count + plsc.all_reduce_population_count(masks[i])[0]
        return count

    count = lax.fori_loop(0, _E_PER_W // (16 * _UNROLL), grp_body,
                          jnp.int32(0))

    vpad = jnp.full((16,), _N_TAILS * 16384 + _N_NODES, jnp.int32)
    for t in range(8):
        cp_v[pl.ds(count + 16 * t, 16)] = vpad
    ngroups = (count + 127) // 128

    def g_body(g, carry):
        for i in range(8):
            s = pl.ds(i * 16, 16)
            v16 = cp_v[pl.ds(g * 128 + i * 16, 16)]
            cs_row[s] = lax.bitwise_and(v16, 16383)
            cj_row[s] = lax.shift_right_logical(v16, 14)
        pltpu.async_copy(wp_hbm.at[cs_row], rows_g, sem_g).wait()
        pltpu.sync_copy(rows_g, acc_sh.at[cj_row], add=True)
        return carry

    lax.fori_loop(0, ngroups, g_body, 0)
    plsc.subcore_barrier()
    pltpu.sync_copy(acc_sh.at[pl.ds(sid * _ZROWS, _ZROWS)],
                    out_hbm.at[cid, pl.ds(sid * _ZROWS, _ZROWS)])


_edge_stage = functools.partial(
    pl.kernel,
    out_type=jax.ShapeDtypeStruct((_NC, _A_ROWS, _PPAD), jnp.float32),
    mesh=plsc.VectorSubcoreMesh(core_axis_name="c", subcore_axis_name="s"),
    scratch_types=[
        pltpu.VMEM((_SLOT_PAD,), jnp.int32),
        pltpu.VMEM((_E_PER_W,), jnp.int32),
        pltpu.VMEM((_E_PER_W,), jnp.int32),
        pltpu.VMEM((_E_PER_W,), jnp.int32),
        pltpu.VMEM((_CAP,), jnp.int32),
        pltpu.VMEM((128,), jnp.int32),
        pltpu.VMEM((128,), jnp.int32),
        pltpu.VMEM((128, _PPAD), jnp.float32),
        pltpu.VMEM_SHARED((_A_ROWS, _PPAD), jnp.float32),
        pltpu.SemaphoreType.DMA,
        pltpu.SemaphoreType.DMA,
        pltpu.SemaphoreType.DMA,
    ],
    compiler_params=pltpu.CompilerParams(
        needs_layout_passes=False, use_tc_tiling_on_sc=False),
)(_edge_stage_kernel)


def _final_stage_kernel(a_hbm, m_ref, proto_ref, slot_ref, out_ref, a_vmem,
                        sem_a):
    pltpu.async_copy(a_hbm, a_vmem, sem_a).wait()
    a = a_vmem[0] + a_vmem[1]
    qrows = _STRIDE // 4
    slot = slot_ref[...]
    cols = lax.broadcasted_iota(jnp.int32, (_N_TAILS, qrows), 1)
    q = lax.shift_right_logical(slot, 2)
    u_b = lax.bitwise_and(slot, 3)
    g = jnp.zeros((_N_TAILS, _HIDDEN), jnp.float32)
    for u in range(4):
        acc = jnp.zeros((qrows, _HIDDEN), jnp.float32)
        for r in range(_NUM_REL):
            slab = lax.slice(a, (r * qrows, u * _PPAD),
                             ((r + 1) * qrows, (u + 1) * _PPAD))
            acc = acc + lax.dot_general(
                slab, m_ref[r], (((1,), (0,)), ((), ())),
                precision=lax.Precision.DEFAULT,
                preferred_element_type=jnp.float32)
        oh = ((cols == q) & (u_b == u)).astype(jnp.float32)
        g = g + lax.dot_general(
            oh, acc, (((1,), (0,)), ((), ())),
            precision=lax.Precision.DEFAULT,
            preferred_element_type=jnp.float32)
    gn = g / jnp.maximum(
        jnp.sqrt(jnp.sum(g * g, axis=1, keepdims=True)), 1e-12)
    pr = proto_ref[...]
    pn = pr / jnp.maximum(
        jnp.sqrt(jnp.sum(pr * pr, axis=1, keepdims=True)), 1e-12)
    out_ref[...] = lax.dot_general(
        gn, pn, (((1,), (1,)), ((), ())),
        precision=lax.Precision.HIGHEST, preferred_element_type=jnp.float32)


def kernel(input, multi_embedding, proto_embedding, edge_index, edge_type,
           tail_indices):
    x = input.astype(jnp.float32)
    ei = edge_index.astype(jnp.int32)
    et = edge_type.astype(jnp.int32)
    tails = tail_indices.astype(jnp.int32)

    proto_p = jnp.zeros((_PPAD, _HIDDEN), jnp.float32)
    proto_p = proto_p.at[:_NUM_PROTO].set(proto_embedding.astype(jnp.float32))

    wp = pl.pallas_call(
        _node_stage_kernel,
        out_shape=jax.ShapeDtypeStruct((_WP_ROWS, _PPAD), jnp.float32),
        grid=(_WP_ROWS // _NODE_BLK,),
        in_specs=[
            pl.BlockSpec((_NODE_BLK, _HIDDEN), lambda i: (i, 0)),
            pl.BlockSpec((_PPAD, _HIDDEN), lambda i: (0, 0)),
        ],
        out_specs=pl.BlockSpec((_NODE_BLK, _PPAD), lambda i: (i, 0)),
    )(x, proto_p)

    slot_tab = jnp.full((_SLOT_PAD,), _N_TAILS, jnp.int32)
    slot_tab = slot_tab.at[tails].set(jnp.arange(_N_TAILS, dtype=jnp.int32))
    zeros_hbm = jnp.zeros((_NS, _ZROWS, _PPAD), jnp.float32)

    a_out = _edge_stage(ei, et, slot_tab, wp, zeros_hbm)

    m_pad = jnp.zeros((_NUM_REL, _PPAD, _HIDDEN), jnp.float32)
    m_pad = m_pad.at[:, :_NUM_PROTO, :].set(
        multi_embedding.astype(jnp.float32))
    slot_t = slot_tab[tails].reshape(_N_TAILS, 1)

    out = pl.pallas_call(
        _final_stage_kernel,
        out_shape=jax.ShapeDtypeStruct((_N_TAILS, _PPAD), jnp.float32),
        in_specs=[
            pl.BlockSpec(memory_space=pl.ANY),
            pl.BlockSpec(memory_space=pltpu.VMEM),
            pl.BlockSpec(memory_space=pltpu.VMEM),
            pl.BlockSpec(memory_space=pltpu.VMEM),
        ],
        scratch_shapes=[
            pltpu.VMEM((_NC, _A_ROWS // 4, 128), jnp.float32),
            pltpu.SemaphoreType.DMA,
        ],
    )(a_out.reshape(_NC, _A_ROWS // 4, 128), m_pad, proto_p, slot_t)
    return out[:, :_NUM_PROTO]
